# baseline (speedup 1.0000x reference)
import functools

import jax
import jax.numpy as jnp
from jax.experimental import pallas as pl
from jax.experimental.pallas import tpu as pltpu

TEMPERATURE = 1.0


def _router_body(hid_ref, w1_ref, b1_ref, w2_ref, b2_ref, gl_ref, qb_ref,
                 ids_ref, w_ref, bias_ref, wexp_ref, acc_ref,
                 *, B, S, K, MEM, NUM_SLOTS):
    st = pl.program_id(0)
    nst = pl.num_programs(0)

    @pl.when(st == 0)
    def _():
        acc_ref[...] = jnp.zeros_like(acc_ref)

    acc_ref[...] += jnp.sum(hid_ref[...], axis=1)

    @pl.when(st == nst - 1)
    def _():
        q = acc_ref[...] * (1.0 / S)
        h = jnp.maximum(
            jnp.dot(q, w1_ref[...], preferred_element_type=jnp.float32)
            + b1_ref[...], 0.0)
        logits = (jnp.dot(h, w2_ref[...], preferred_element_type=jnp.float32)
                  + b2_ref[...]) / TEMPERATURE
        iota_ns = jax.lax.broadcasted_iota(jnp.int32, (B, NUM_SLOTS), 1)
        l = logits
        m_list, idx_list = [], []
        for _ in range(K):
            m = jnp.max(l, axis=1, keepdims=True)
            idx = jnp.min(jnp.where(l == m, iota_ns, NUM_SLOTS),
                          axis=1, keepdims=True)
            m_list.append(m)
            idx_list.append(idx)
            l = jnp.where(iota_ns == idx, -1e30, l)
        e_list = [jnp.exp(m - m_list[0]) for m in m_list]
        esum = e_list[0]
        for e in e_list[1:]:
            esum = esum + e
        w_list = [e / esum for e in e_list]
        ck = jax.lax.broadcasted_iota(jnp.int32, (B, K), 1)
        ids_out = jnp.zeros((B, K), jnp.int32)
        w_out = jnp.zeros((B, K), jnp.float32)
        for j in range(K):
            sel = (ck == j)
            ids_out = ids_out + jnp.where(sel, idx_list[j], 0)
            w_out = w_out + jnp.where(sel, w_list[j], 0.0)
        ids_ref[...] = ids_out
        w_ref[...] = w_out
        tbl = gl_ref[...] + qb_ref[...]
        KM = K * MEM
        colc = jax.lax.broadcasted_iota(jnp.int32, (MEM, KM), 1)
        rowc = jax.lax.broadcasted_iota(jnp.int32, (MEM, KM), 0)
        bias_out = jnp.zeros((B, KM), jnp.float32)
        for j in range(K):
            oh = (iota_ns == idx_list[j]).astype(jnp.float32)
            bj = jnp.dot(oh, tbl, preferred_element_type=jnp.float32)
            selc = (colc == rowc + j * MEM).astype(jnp.float32)
            bias_out = bias_out + jnp.dot(
                bj, selc, preferred_element_type=jnp.float32)
        bias_ref[...] = bias_out
        ckm = jax.lax.broadcasted_iota(jnp.int32, (B, KM), 1) // MEM
        wexp_out = jnp.zeros((B, KM), jnp.float32)
        for j in range(K):
            wexp_out = wexp_out + jnp.where(ckm == j, w_list[j], 0.0)
        wexp_ref[...] = wexp_out


def _main_body(slot_ref, *refs, B, K, MEM, D, TS):
    hid_ref = refs[0]
    qw_refs = refs[1:1 + K]
    mem_refs = refs[1 + K:1 + 2 * K]
    bias_ref = refs[1 + 2 * K]
    wexp_ref = refs[2 + 2 * K]
    out_ref = refs[3 + 2 * K]
    qwc_s = refs[4 + 2 * K]
    memc_s = refs[5 + 2 * K]
    st = pl.program_id(1)
    KM = K * MEM

    @pl.when(st == 0)
    def _():
        colc = jax.lax.broadcasted_iota(jnp.int32, (MEM, KM), 1)
        rowc = jax.lax.broadcasted_iota(jnp.int32, (MEM, KM), 0)
        qwc_s[...] = jnp.zeros((D, KM), jnp.float32)
        for j in range(K):
            selc = (colc == rowc + j * MEM).astype(jnp.float32)
            qwc_s[...] += jnp.dot(qw_refs[j][0], selc,
                                  preferred_element_type=jnp.float32)
            memc_s[j * MEM:(j + 1) * MEM, :] = mem_refs[j][0]

    h = hid_ref[0]
    scores = jnp.dot(h, qwc_s[...], preferred_element_type=jnp.float32)
    g = jax.nn.sigmoid(scores + bias_ref[0]) * wexp_ref[0]
    out_ref[0] = h + jnp.dot(g, memc_s[...],
                             preferred_element_type=jnp.float32)


def kernel(hidden_states, W1, b1, W2, b2, memory, gate_logits, qW, qb, top_k):
    B, S, D = hidden_states.shape
    NUM_SLOTS, MEM, _ = memory.shape
    RH = W1.shape[1]
    K = 8
    KM = K * MEM
    TSR = 128
    TS = 256

    router = pl.pallas_call(
        functools.partial(_router_body, B=B, S=S, K=K, MEM=MEM,
                          NUM_SLOTS=NUM_SLOTS),
        grid=(S // TSR,),
        in_specs=[
            pl.BlockSpec((B, TSR, D), lambda i: (0, i, 0)),
            pl.BlockSpec((D, RH), lambda i: (0, 0)),
            pl.BlockSpec((1, RH), lambda i: (0, 0)),
            pl.BlockSpec((RH, NUM_SLOTS), lambda i: (0, 0)),
            pl.BlockSpec((1, NUM_SLOTS), lambda i: (0, 0)),
            pl.BlockSpec((NUM_SLOTS, MEM), lambda i: (0, 0)),
            pl.BlockSpec((NUM_SLOTS, MEM), lambda i: (0, 0)),
        ],
        out_specs=[
            pl.BlockSpec((B, K), lambda i: (0, 0)),
            pl.BlockSpec((B, K), lambda i: (0, 0)),
            pl.BlockSpec((B, KM), lambda i: (0, 0)),
            pl.BlockSpec((B, KM), lambda i: (0, 0)),
        ],
        out_shape=[
            jax.ShapeDtypeStruct((B, K), jnp.int32),
            jax.ShapeDtypeStruct((B, K), jnp.float32),
            jax.ShapeDtypeStruct((B, KM), jnp.float32),
            jax.ShapeDtypeStruct((B, KM), jnp.float32),
        ],
        scratch_shapes=[pltpu.VMEM((B, D), jnp.float32)],
    )
    slot_ids, weights, bias_cat, wexp = router(
        hidden_states, W1, b1.reshape(1, RH), W2, b2.reshape(1, NUM_SLOTS),
        gate_logits, qb)

    slot_flat = slot_ids.reshape(-1)
    bias3 = bias_cat.reshape(B, 1, KM)
    wexp3 = wexp.reshape(B, 1, KM)

    def qw_idx(j):
        return lambda b, st, sref: (sref[b * K + j], 0, 0)

    grid_spec = pltpu.PrefetchScalarGridSpec(
        num_scalar_prefetch=1,
        grid=(B, S // TS),
        in_specs=[
            pl.BlockSpec((1, TS, D), lambda b, st, sref: (b, st, 0)),
            *[pl.BlockSpec((1, D, MEM), qw_idx(j)) for j in range(K)],
            *[pl.BlockSpec((1, MEM, D), qw_idx(j)) for j in range(K)],
            pl.BlockSpec((1, 1, KM), lambda b, st, sref: (b, 0, 0)),
            pl.BlockSpec((1, 1, KM), lambda b, st, sref: (b, 0, 0)),
        ],
        out_specs=pl.BlockSpec((1, TS, D), lambda b, st, sref: (b, st, 0)),
        scratch_shapes=[
            pltpu.VMEM((D, KM), jnp.float32),
            pltpu.VMEM((KM, D), jnp.float32),
        ],
    )
    main = pl.pallas_call(
        functools.partial(_main_body, B=B, K=K, MEM=MEM, D=D, TS=TS),
        grid_spec=grid_spec,
        out_shape=jax.ShapeDtypeStruct((B, S, D), jnp.float32),
    )
    out = main(slot_flat, hidden_states,
               *([qW] * K), *([memory] * K), bias3, wexp3)
    return out, slot_ids, weights


# speedup vs baseline: 1.1029x; 1.1029x over previous
import functools

import jax
import jax.numpy as jnp
from jax.experimental import pallas as pl
from jax.experimental.pallas import tpu as pltpu

TEMPERATURE = 1.0


def _fused_body(hid_ref, w1_ref, b1_ref, w2_ref, b2_ref, gl_ref, qb_ref,
                qw_any, mem_any,
                out_ref, ids_ref, w_ref,
                qwsc, qwc_s, memc_s, bias_s, wexp_s, sem,
                *, S, D, K, MEM, NUM_SLOTS, TS):
    st = pl.program_id(1)
    KM = K * MEM

    @pl.when(st == 0)
    def _():
        h_all = hid_ref[0]
        q = jnp.sum(h_all, axis=0, keepdims=True) * (1.0 / S)
        hmlp = jnp.maximum(
            jnp.dot(q, w1_ref[...], preferred_element_type=jnp.float32)
            + b1_ref[...], 0.0)
        logits = (jnp.dot(hmlp, w2_ref[...],
                          preferred_element_type=jnp.float32)
                  + b2_ref[...]) / TEMPERATURE
        iota_ns = jax.lax.broadcasted_iota(jnp.int32, (1, NUM_SLOTS), 1)
        l = logits
        m_list, idx_list = [], []
        for _ in range(K):
            m = jnp.max(l)
            idx = jnp.min(jnp.where(l == m, iota_ns, NUM_SLOTS))
            m_list.append(m)
            idx_list.append(idx)
            l = jnp.where(iota_ns == idx, -1e30, l)
        copies = []
        for j in range(K):
            copies.append(pltpu.make_async_copy(
                qw_any.at[idx_list[j]], qwsc.at[j], sem))
            copies.append(pltpu.make_async_copy(
                mem_any.at[idx_list[j]], memc_s.at[pl.ds(j * MEM, MEM), :],
                sem))
        for c in copies:
            c.start()
        e_list = [jnp.exp(m - m_list[0]) for m in m_list]
        esum = e_list[0]
        for e in e_list[1:]:
            esum = esum + e
        w_list = [e / esum for e in e_list]
        ck = jax.lax.broadcasted_iota(jnp.int32, (1, 1, K), 2)
        ids_out = jnp.zeros((1, 1, K), jnp.int32)
        w_out = jnp.zeros((1, 1, K), jnp.float32)
        for j in range(K):
            ids_out = jnp.where(ck == j, idx_list[j], ids_out)
            w_out = jnp.where(ck == j, w_list[j], w_out)
        ids_ref[...] = ids_out
        w_ref[...] = w_out
        tbl = gl_ref[...] + qb_ref[...]
        colc = jax.lax.broadcasted_iota(jnp.int32, (MEM, KM), 1)
        rowc = jax.lax.broadcasted_iota(jnp.int32, (MEM, KM), 0)
        bias_out = jnp.zeros((1, KM), jnp.float32)
        wexp_out = jnp.zeros((1, KM), jnp.float32)
        ckm = jax.lax.broadcasted_iota(jnp.int32, (1, KM), 1) // MEM
        for j in range(K):
            oh = (iota_ns == idx_list[j]).astype(jnp.float32)
            bj = jnp.dot(oh, tbl, preferred_element_type=jnp.float32)
            selc = (colc == rowc + j * MEM).astype(jnp.float32)
            bias_out = bias_out + jnp.dot(
                bj, selc, preferred_element_type=jnp.float32)
            wexp_out = jnp.where(ckm == j, w_list[j], wexp_out)
        bias_s[...] = bias_out
        wexp_s[...] = wexp_out
        for c in copies:
            c.wait()
        qwc_s[...] = jnp.zeros((D, KM), jnp.float32)
        for j in range(K):
            selc = (colc == rowc + j * MEM).astype(jnp.float32)
            qwc_s[...] += jnp.dot(qwsc[j], selc,
                                  preferred_element_type=jnp.float32)

    h = hid_ref[0, pl.ds(st * TS, TS), :]
    scores = jnp.dot(h, qwc_s[...], preferred_element_type=jnp.float32)
    g = jax.nn.sigmoid(scores + bias_s[...]) * wexp_s[...]
    out_ref[0] = h + jnp.dot(g, memc_s[...],
                             preferred_element_type=jnp.float32)


def kernel(hidden_states, W1, b1, W2, b2, memory, gate_logits, qW, qb, top_k):
    B, S, D = hidden_states.shape
    NUM_SLOTS, MEM, _ = memory.shape
    RH = W1.shape[1]
    K = 8
    KM = K * MEM
    TS = 256

    fused = pl.pallas_call(
        functools.partial(_fused_body, S=S, D=D, K=K, MEM=MEM,
                          NUM_SLOTS=NUM_SLOTS, TS=TS),
        grid=(B, S // TS),
        in_specs=[
            pl.BlockSpec((1, S, D), lambda b, st: (b, 0, 0)),
            pl.BlockSpec((D, RH), lambda b, st: (0, 0)),
            pl.BlockSpec((1, RH), lambda b, st: (0, 0)),
            pl.BlockSpec((RH, NUM_SLOTS), lambda b, st: (0, 0)),
            pl.BlockSpec((1, NUM_SLOTS), lambda b, st: (0, 0)),
            pl.BlockSpec((NUM_SLOTS, MEM), lambda b, st: (0, 0)),
            pl.BlockSpec((NUM_SLOTS, MEM), lambda b, st: (0, 0)),
            pl.BlockSpec(memory_space=pl.ANY),
            pl.BlockSpec(memory_space=pl.ANY),
        ],
        out_specs=[
            pl.BlockSpec((1, TS, D), lambda b, st: (b, st, 0)),
            pl.BlockSpec((1, 1, K), lambda b, st: (b, 0, 0)),
            pl.BlockSpec((1, 1, K), lambda b, st: (b, 0, 0)),
        ],
        out_shape=[
            jax.ShapeDtypeStruct((B, S, D), jnp.float32),
            jax.ShapeDtypeStruct((B, 1, K), jnp.int32),
            jax.ShapeDtypeStruct((B, 1, K), jnp.float32),
        ],
        scratch_shapes=[
            pltpu.VMEM((K, D, MEM), jnp.float32),
            pltpu.VMEM((D, KM), jnp.float32),
            pltpu.VMEM((KM, D), jnp.float32),
            pltpu.VMEM((1, KM), jnp.float32),
            pltpu.VMEM((1, KM), jnp.float32),
            pltpu.SemaphoreType.DMA,
        ],
        compiler_params=pltpu.CompilerParams(
            dimension_semantics=("arbitrary", "arbitrary")),
    )
    out, ids3, w3 = fused(
        hidden_states, W1, b1.reshape(1, RH), W2, b2.reshape(1, NUM_SLOTS),
        gate_logits, qb, qW, memory)
    return out, ids3.reshape(B, K), w3.reshape(B, K)


# speedup vs baseline: 1.1581x; 1.0501x over previous
import functools

import jax
import jax.numpy as jnp
from jax.experimental import pallas as pl
from jax.experimental.pallas import tpu as pltpu

TEMPERATURE = 1.0


def _fused_body(hid_ref, w1_ref, b1_ref, w2_ref, b2_ref, gl_ref, qb_ref,
                qw_any, mem_any,
                out_ref, ids_ref, w_ref,
                qwsc, qwc_s, memc_s, memc_bf, bias_s, wexp_s, sem,
                *, S, D, K, MEM, NUM_SLOTS, TS):
    st = pl.program_id(1)
    KM = K * MEM

    @pl.when(st == 0)
    def _():
        h_all = hid_ref[0]
        q = jnp.sum(h_all, axis=0, keepdims=True) * (1.0 / S)
        hmlp = jnp.maximum(
            jnp.dot(q, w1_ref[...], preferred_element_type=jnp.float32)
            + b1_ref[...], 0.0)
        logits = (jnp.dot(hmlp, w2_ref[...],
                          preferred_element_type=jnp.float32)
                  + b2_ref[...]) / TEMPERATURE
        iota_ns = jax.lax.broadcasted_iota(jnp.int32, (1, NUM_SLOTS), 1)
        l = logits
        m_list, idx_list = [], []
        for _ in range(K):
            m = jnp.max(l)
            idx = jnp.min(jnp.where(l == m, iota_ns, NUM_SLOTS))
            m_list.append(m)
            idx_list.append(idx)
            l = jnp.where(iota_ns == idx, -1e30, l)
        copies = []
        for j in range(K):
            copies.append(pltpu.make_async_copy(
                qw_any.at[idx_list[j]], qwsc.at[j], sem))
            copies.append(pltpu.make_async_copy(
                mem_any.at[idx_list[j]], memc_s.at[pl.ds(j * MEM, MEM), :],
                sem))
        for c in copies:
            c.start()
        e_list = [jnp.exp(m - m_list[0]) for m in m_list]
        esum = e_list[0]
        for e in e_list[1:]:
            esum = esum + e
        w_list = [e / esum for e in e_list]
        ck = jax.lax.broadcasted_iota(jnp.int32, (1, 1, K), 2)
        ids_out = jnp.zeros((1, 1, K), jnp.int32)
        w_out = jnp.zeros((1, 1, K), jnp.float32)
        for j in range(K):
            ids_out = jnp.where(ck == j, idx_list[j], ids_out)
            w_out = jnp.where(ck == j, w_list[j], w_out)
        ids_ref[...] = ids_out
        w_ref[...] = w_out
        tbl = gl_ref[...] + qb_ref[...]
        colc = jax.lax.broadcasted_iota(jnp.int32, (MEM, KM), 1)
        rowc = jax.lax.broadcasted_iota(jnp.int32, (MEM, KM), 0)
        bias_out = jnp.zeros((1, KM), jnp.float32)
        wexp_out = jnp.zeros((1, KM), jnp.float32)
        ckm = jax.lax.broadcasted_iota(jnp.int32, (1, KM), 1) // MEM
        for j in range(K):
            oh = (iota_ns == idx_list[j]).astype(jnp.float32)
            bj = jnp.dot(oh, tbl, preferred_element_type=jnp.float32)
            selc = (colc == rowc + j * MEM).astype(jnp.float32)
            bias_out = bias_out + jnp.dot(
                bj, selc, preferred_element_type=jnp.float32)
            wexp_out = jnp.where(ckm == j, w_list[j], wexp_out)
        bias_s[...] = bias_out
        wexp_s[...] = wexp_out
        for c in copies:
            c.wait()
        qwc_s[...] = jnp.zeros((D, KM), jnp.bfloat16)
        for j in range(K):
            selc = (colc == rowc + j * MEM).astype(jnp.bfloat16)
            qwc_s[...] += jnp.dot(qwsc[j].astype(jnp.bfloat16), selc,
                                  preferred_element_type=jnp.float32
                                  ).astype(jnp.bfloat16)
        memc_bf[...] = memc_s[...].astype(jnp.bfloat16)

    h = hid_ref[0, pl.ds(st * TS, TS), :]
    scores = jnp.dot(h.astype(jnp.bfloat16), qwc_s[...],
                     preferred_element_type=jnp.float32)
    g = jax.nn.sigmoid(scores + bias_s[...]) * wexp_s[...]
    out_ref[0] = h + jnp.dot(g.astype(jnp.bfloat16), memc_bf[...],
                             preferred_element_type=jnp.float32)


def kernel(hidden_states, W1, b1, W2, b2, memory, gate_logits, qW, qb, top_k):
    B, S, D = hidden_states.shape
    NUM_SLOTS, MEM, _ = memory.shape
    RH = W1.shape[1]
    K = 8
    KM = K * MEM
    TS = 256

    fused = pl.pallas_call(
        functools.partial(_fused_body, S=S, D=D, K=K, MEM=MEM,
                          NUM_SLOTS=NUM_SLOTS, TS=TS),
        grid=(B, S // TS),
        in_specs=[
            pl.BlockSpec((1, S, D), lambda b, st: (b, 0, 0)),
            pl.BlockSpec((D, RH), lambda b, st: (0, 0)),
            pl.BlockSpec((1, RH), lambda b, st: (0, 0)),
            pl.BlockSpec((RH, NUM_SLOTS), lambda b, st: (0, 0)),
            pl.BlockSpec((1, NUM_SLOTS), lambda b, st: (0, 0)),
            pl.BlockSpec((NUM_SLOTS, MEM), lambda b, st: (0, 0)),
            pl.BlockSpec((NUM_SLOTS, MEM), lambda b, st: (0, 0)),
            pl.BlockSpec(memory_space=pl.ANY),
            pl.BlockSpec(memory_space=pl.ANY),
        ],
        out_specs=[
            pl.BlockSpec((1, TS, D), lambda b, st: (b, st, 0)),
            pl.BlockSpec((1, 1, K), lambda b, st: (b, 0, 0)),
            pl.BlockSpec((1, 1, K), lambda b, st: (b, 0, 0)),
        ],
        out_shape=[
            jax.ShapeDtypeStruct((B, S, D), jnp.float32),
            jax.ShapeDtypeStruct((B, 1, K), jnp.int32),
            jax.ShapeDtypeStruct((B, 1, K), jnp.float32),
        ],
        scratch_shapes=[
            pltpu.VMEM((K, D, MEM), jnp.float32),
            pltpu.VMEM((D, KM), jnp.bfloat16),
            pltpu.VMEM((KM, D), jnp.float32),
            pltpu.VMEM((KM, D), jnp.bfloat16),
            pltpu.VMEM((1, KM), jnp.float32),
            pltpu.VMEM((1, KM), jnp.float32),
            pltpu.SemaphoreType.DMA,
        ],
        compiler_params=pltpu.CompilerParams(
            dimension_semantics=("arbitrary", "arbitrary")),
    )
    out, ids3, w3 = fused(
        hidden_states, W1, b1.reshape(1, RH), W2, b2.reshape(1, NUM_SLOTS),
        gate_logits, qb, qW, memory)
    return out, ids3.reshape(B, K), w3.reshape(B, K)


# speedup vs baseline: 1.2066x; 1.0419x over previous
import functools

import jax
import jax.numpy as jnp
from jax.experimental import pallas as pl
from jax.experimental.pallas import tpu as pltpu

TEMPERATURE = 1.0


def _fused_body(hid_ref, w1_ref, b1_ref, w2_ref, b2_ref, gl_ref, qb_ref,
                qw_any, mem_any,
                out_ref, ids_ref, w_ref,
                hsc, acc_s, qwsc, qwc_s, memc_s, memc_bf, bias_s, wexp_s, sem,
                *, B, S, D, K, MEM, NUM_SLOTS, TS, NST):
    bp = pl.program_id(0)
    st = pl.program_id(1)
    KM = K * MEM

    @pl.when(bp < B)
    def _fill():
        par = jax.lax.rem(bp, 2)
        h_t = hid_ref[0]
        hsc[par, pl.ds(st * TS, TS), :] = h_t
        psum = jnp.sum(h_t, axis=0, keepdims=True)

        @pl.when(st == 0)
        def _():
            acc_s[...] = psum

        @pl.when(st > 0)
        def _():
            acc_s[...] += psum

        @pl.when(st == NST - 1)
        def _route():
            q = acc_s[...] * (1.0 / S)
            hmlp = jnp.maximum(
                jnp.dot(q, w1_ref[...], preferred_element_type=jnp.float32)
                + b1_ref[...], 0.0)
            logits = (jnp.dot(hmlp, w2_ref[...],
                              preferred_element_type=jnp.float32)
                      + b2_ref[...]) / TEMPERATURE
            iota_ns = jax.lax.broadcasted_iota(jnp.int32, (1, NUM_SLOTS), 1)
            l = logits
            m_list, idx_list = [], []
            for _ in range(K):
                m = jnp.max(l)
                idx = jnp.min(jnp.where(l == m, iota_ns, NUM_SLOTS))
                m_list.append(m)
                idx_list.append(idx)
                l = jnp.where(iota_ns == idx, -1e30, l)
            copies = []
            for j in range(K):
                copies.append(pltpu.make_async_copy(
                    qw_any.at[idx_list[j]], qwsc.at[j], sem))
                copies.append(pltpu.make_async_copy(
                    mem_any.at[idx_list[j]],
                    memc_s.at[par, pl.ds(j * MEM, MEM), :], sem))
            for c in copies:
                c.start()
            e_list = [jnp.exp(m - m_list[0]) for m in m_list]
            esum = e_list[0]
            for e in e_list[1:]:
                esum = esum + e
            w_list = [e / esum for e in e_list]
            ck = jax.lax.broadcasted_iota(jnp.int32, (1, 1, K), 2)
            ids_out = jnp.zeros((1, 1, K), jnp.int32)
            w_out = jnp.zeros((1, 1, K), jnp.float32)
            for j in range(K):
                ids_out = jnp.where(ck == j, idx_list[j], ids_out)
                w_out = jnp.where(ck == j, w_list[j], w_out)
            ids_ref[...] = ids_out
            w_ref[...] = w_out
            tbl = gl_ref[...] + qb_ref[...]
            colc = jax.lax.broadcasted_iota(jnp.int32, (MEM, KM), 1)
            rowc = jax.lax.broadcasted_iota(jnp.int32, (MEM, KM), 0)
            bias_out = jnp.zeros((1, KM), jnp.float32)
            wexp_out = jnp.zeros((1, KM), jnp.float32)
            ckm = jax.lax.broadcasted_iota(jnp.int32, (1, KM), 1) // MEM
            for j in range(K):
                oh = (iota_ns == idx_list[j]).astype(jnp.float32)
                bj = jnp.dot(oh, tbl, preferred_element_type=jnp.float32)
                selc = (colc == rowc + j * MEM).astype(jnp.float32)
                bias_out = bias_out + jnp.dot(
                    bj, selc, preferred_element_type=jnp.float32)
                wexp_out = jnp.where(ckm == j, w_list[j], wexp_out)
            bias_s[par] = bias_out
            wexp_s[par] = wexp_out
            for c in copies:
                c.wait()
            qwc_s[par] = jnp.zeros((D, KM), jnp.bfloat16)
            for j in range(K):
                selc = (colc == rowc + j * MEM).astype(jnp.bfloat16)
                qwc_s[par] += jnp.dot(qwsc[j].astype(jnp.bfloat16), selc,
                                      preferred_element_type=jnp.float32
                                      ).astype(jnp.bfloat16)
            memc_bf[par] = memc_s[par].astype(jnp.bfloat16)

    @pl.when(bp >= 1)
    def _compute():
        pac = jax.lax.rem(bp - 1, 2)
        h = hsc[pac, pl.ds(st * TS, TS), :]
        scores = jnp.dot(h.astype(jnp.bfloat16), qwc_s[pac],
                         preferred_element_type=jnp.float32)
        g = jax.nn.sigmoid(scores + bias_s[pac]) * wexp_s[pac]
        out_ref[0] = h + jnp.dot(g.astype(jnp.bfloat16), memc_bf[pac],
                                 preferred_element_type=jnp.float32)


def kernel(hidden_states, W1, b1, W2, b2, memory, gate_logits, qW, qb, top_k):
    B, S, D = hidden_states.shape
    NUM_SLOTS, MEM, _ = memory.shape
    RH = W1.shape[1]
    K = 8
    KM = K * MEM
    TS = 256
    NST = S // TS

    fused = pl.pallas_call(
        functools.partial(_fused_body, B=B, S=S, D=D, K=K, MEM=MEM,
                          NUM_SLOTS=NUM_SLOTS, TS=TS, NST=NST),
        grid=(B + 1, NST),
        in_specs=[
            pl.BlockSpec((1, TS, D),
                         lambda bp, st: (jnp.minimum(bp, B - 1),
                                         jnp.where(bp < B, st, NST - 1), 0)),
            pl.BlockSpec((D, RH), lambda bp, st: (0, 0)),
            pl.BlockSpec((1, RH), lambda bp, st: (0, 0)),
            pl.BlockSpec((RH, NUM_SLOTS), lambda bp, st: (0, 0)),
            pl.BlockSpec((1, NUM_SLOTS), lambda bp, st: (0, 0)),
            pl.BlockSpec((NUM_SLOTS, MEM), lambda bp, st: (0, 0)),
            pl.BlockSpec((NUM_SLOTS, MEM), lambda bp, st: (0, 0)),
            pl.BlockSpec(memory_space=pl.ANY),
            pl.BlockSpec(memory_space=pl.ANY),
        ],
        out_specs=[
            pl.BlockSpec((1, TS, D),
                         lambda bp, st: (jnp.maximum(bp - 1, 0),
                                         jnp.where(bp >= 1, st, 0), 0)),
            pl.BlockSpec((1, 1, K),
                         lambda bp, st: (jnp.minimum(bp, B - 1), 0, 0)),
            pl.BlockSpec((1, 1, K),
                         lambda bp, st: (jnp.minimum(bp, B - 1), 0, 0)),
        ],
        out_shape=[
            jax.ShapeDtypeStruct((B, S, D), jnp.float32),
            jax.ShapeDtypeStruct((B, 1, K), jnp.int32),
            jax.ShapeDtypeStruct((B, 1, K), jnp.float32),
        ],
        scratch_shapes=[
            pltpu.VMEM((2, S, D), jnp.float32),
            pltpu.VMEM((1, D), jnp.float32),
            pltpu.VMEM((K, D, MEM), jnp.float32),
            pltpu.VMEM((2, D, KM), jnp.bfloat16),
            pltpu.VMEM((2, KM, D), jnp.float32),
            pltpu.VMEM((2, KM, D), jnp.bfloat16),
            pltpu.VMEM((2, 1, KM), jnp.float32),
            pltpu.VMEM((2, 1, KM), jnp.float32),
            pltpu.SemaphoreType.DMA,
        ],
        compiler_params=pltpu.CompilerParams(
            dimension_semantics=("arbitrary", "arbitrary")),
    )
    out, ids3, w3 = fused(
        hidden_states, W1, b1.reshape(1, RH), W2, b2.reshape(1, NUM_SLOTS),
        gate_logits, qb, qW, memory)
    return out, ids3.reshape(B, K), w3.reshape(B, K)
